# baseline (speedup 1.0000x reference)
import jax
import jax.numpy as jnp
from jax.experimental import pallas as pl
from jax.experimental.pallas import tpu as pltpu

_NEG = -1e30
_TDK = 512
_TS = 1024
_WIN = 72


def _fused_kernel(lens_ref, sel_ref, h_ref, w_ref, ctx_ref,
                  attn_ref, wout_ref,
                  tgtc_ref, acc_ref, win_ref, aw_ref, scl_ref):
    d = pl.program_id(0)
    j = pl.program_id(1)
    nd = pl.num_programs(0)
    nj = pl.num_programs(1)
    batch = h_ref.shape[0]
    ns = nj // batch
    b = j // ns
    s = j % ns
    nblk = lens_ref.shape[1]

    @pl.when(j == 0)
    def _w_step():
        tgtc_ref[...] = jax.lax.dot_general(
            h_ref[...], w_ref[...], (((1,), (1,)), ((), ())),
            preferred_element_type=jnp.float32)

    ctx = ctx_ref[0]
    tgt_b = tgtc_ref[pl.ds(b, 1), :]
    part = jax.lax.dot_general(
        tgt_b, ctx, (((1,), (1,)), ((), ())),
        preferred_element_type=jnp.float32)

    @pl.when(d == 0)
    def _acc_init():
        acc_ref[pl.ds(j, 1), 0, :] = part

    @pl.when(d > 0)
    def _acc_add():
        acc_ref[pl.ds(j, 1), 0, :] += part

    @pl.when(s == 0)
    def _window():
        @pl.when(d == 0)
        def _scl():
            sel = sel_ref[b]

            def body(k, tot):
                return tot + jnp.where(k < sel, lens_ref[b, k], 0)

            scl_ref[b, 0] = jax.lax.fori_loop(0, nblk, body, 0) + 1
            scl_ref[b, 1] = lens_ref[b, sel]

        sc = scl_ref[b, 0]
        base = pl.multiple_of((sc // 8) * 8, 8)
        rows = ctx_ref[0, pl.ds(base, _WIN), :]
        win_ref[pl.ds(b * nd + d, 1)] = rows[None]
        aw = jax.lax.dot_general(
            tgt_b, rows, (((1,), (1,)), ((), ())),
            preferred_element_type=jnp.float32)

        @pl.when(d == 0)
        def _aw0():
            aw_ref[pl.ds(b, 1), :] = aw

        @pl.when(d > 0)
        def _awn():
            aw_ref[pl.ds(b, 1), :] += aw

    @pl.when(d == nd - 1)
    def _emit_attn():
        attn_ref[0] = acc_ref[pl.ds(j, 1), 0, :]

    @pl.when((d == nd - 1) & (s == ns - 1))
    def _finalize():
        ln = scl_ref[b, 1]
        off = scl_ref[b, 0] % 8
        aw = aw_ref[pl.ds(b, 1), :]
        lane = jax.lax.broadcasted_iota(jnp.int32, (1, _WIN), 1)
        mask = (lane >= off) & (lane < off + ln)
        masked = jnp.where(mask, aw, _NEG)
        m = jnp.max(masked)
        e = jnp.where(mask, jnp.exp(masked - m), 0.0)
        den = jnp.sum(e)
        soft = e / jnp.where(den == 0.0, 1.0, den)
        segs = [
            jax.lax.dot_general(
                soft, win_ref[pl.ds(b * nd + dd, 1)][0],
                (((1,), (0,)), ((), ())),
                preferred_element_type=jnp.float32)
            for dd in range(nd)
        ]
        wout_ref[0] = jnp.concatenate(segs, axis=1)


def kernel(h, context, sub_seq_lengths, selected_block_idx, W):
    batch, dim = h.shape
    seq = context.shape[1]
    nd = dim // _TDK
    ns = seq // _TS
    nj = batch * ns
    lens = sub_seq_lengths.astype(jnp.int32)
    sel = selected_block_idx.astype(jnp.int32)

    attn, weighted = pl.pallas_call(
        _fused_kernel,
        grid=(nd, nj),
        in_specs=[
            pl.BlockSpec(memory_space=pltpu.SMEM),
            pl.BlockSpec(memory_space=pltpu.SMEM),
            pl.BlockSpec((batch, dim), lambda d, j: (0, 0)),
            pl.BlockSpec((_TDK, dim), lambda d, j: (d, 0)),
            pl.BlockSpec((1, _TS, _TDK),
                         lambda d, j: (j // ns, j % ns, d)),
        ],
        out_specs=[
            pl.BlockSpec((1, 1, _TS),
                         lambda d, j: (jnp.where(d == nd - 1, j, 0), 0, 0)),
            pl.BlockSpec((1, 1, dim),
                         lambda d, j: (jnp.where(d == nd - 1, j // ns, 0),
                                       0, 0)),
        ],
        out_shape=[
            jax.ShapeDtypeStruct((nj, 1, _TS), jnp.float32),
            jax.ShapeDtypeStruct((batch, 1, dim), jnp.float32),
        ],
        scratch_shapes=[
            pltpu.VMEM((batch, _TDK), jnp.float32),
            pltpu.VMEM((nj, 1, _TS), jnp.float32),
            pltpu.VMEM((batch * nd, _WIN, _TDK), jnp.float32),
            pltpu.VMEM((batch, _WIN), jnp.float32),
            pltpu.SMEM((batch, 2), jnp.int32),
        ],
    )(lens, sel, h, W, context)
    return (weighted.reshape(batch, dim), attn.reshape(batch, seq))


# speedup vs baseline: 1.5275x; 1.5275x over previous
import jax
import jax.numpy as jnp
from jax.experimental import pallas as pl
from jax.experimental.pallas import tpu as pltpu

_NEG = -1e30
_TD = 512
_TS = 1024


def _fused_kernel(lens_ref, sel_ref, h_ref, w_ref, ctx_ref,
                  attn_ref, wout_ref, tgt_ref, scl_ref, md_ref, acc_ref):
    i = pl.program_id(0)
    dim = h_ref.shape[1]
    nw = dim // _TD
    ts = ctx_ref.shape[1]
    nblk = lens_ref.shape[1]
    nsteps = pl.num_programs(0)

    @pl.when(i < nw)
    def _w_phase():
        tgt_ref[:, pl.ds(i * _TD, _TD)] = jax.lax.dot_general(
            h_ref[...], w_ref[...], (((1,), (1,)), ((), ())),
            preferred_element_type=jnp.float32)

    @pl.when(i >= nw)
    def _ctx_phase():
        j = i - nw
        nctx = nsteps - nw
        batch = h_ref.shape[0]
        ns = nctx // batch
        b = j // ns
        s = j % ns

        @pl.when(s == 0)
        def _init():
            sel = sel_ref[b]

            def body(k, tot):
                return tot + jnp.where(k < sel, lens_ref[b, k], 0)

            scl_ref[0] = jax.lax.fori_loop(0, nblk, body, 0) + 1
            scl_ref[1] = lens_ref[b, sel]
            md_ref[0] = _NEG
            md_ref[1] = 0.0
            acc_ref[...] = jnp.zeros_like(acc_ref)

        ctx = ctx_ref[0]
        tgt = tgt_ref[pl.ds(b, 1), :]
        attn_row = jax.lax.dot_general(
            tgt, ctx, (((1,), (1,)), ((), ())),
            preferred_element_type=jnp.float32)
        attn_ref[0, :, pl.ds(s * ts, ts)] = attn_row

        sc = scl_ref[0]
        ln = scl_ref[1]
        t0 = s * ts
        overlap = (t0 < sc + ln) & (t0 + ts > sc)

        @pl.when(overlap)
        def _update():
            pos = t0 + jax.lax.broadcasted_iota(jnp.int32, (1, ts), 1)
            mask = (pos >= sc) & (pos < sc + ln)
            masked = jnp.where(mask, attn_row, _NEG)
            m_old = md_ref[0]
            m_new = jnp.maximum(m_old, jnp.max(masked))
            scale = jnp.exp(m_old - m_new)
            unnorm = jnp.where(mask, jnp.exp(masked - m_new), 0.0)
            md_ref[0] = m_new
            md_ref[1] = md_ref[1] * scale + jnp.sum(unnorm)
            acc_ref[...] = acc_ref[...] * scale + jax.lax.dot_general(
                unnorm, ctx, (((1,), (0,)), ((), ())),
                preferred_element_type=jnp.float32)

        @pl.when(s == ns - 1)
        def _finalize():
            d = md_ref[1]
            denom = jnp.where(d == 0.0, 1.0, d)
            wout_ref[0] = acc_ref[...] / denom


def kernel(h, context, sub_seq_lengths, selected_block_idx, W):
    batch, dim = h.shape
    seq = context.shape[1]
    nw = dim // _TD
    ns = seq // _TS
    nctx = batch * ns
    lens = sub_seq_lengths.astype(jnp.int32)
    sel = selected_block_idx.astype(jnp.int32)

    attn, weighted = pl.pallas_call(
        _fused_kernel,
        grid=(nw + nctx,),
        in_specs=[
            pl.BlockSpec(memory_space=pltpu.SMEM),
            pl.BlockSpec(memory_space=pltpu.SMEM),
            pl.BlockSpec((batch, dim), lambda i: (0, 0)),
            pl.BlockSpec((_TD, dim), lambda i: (jnp.minimum(i, nw - 1), 0)),
            pl.BlockSpec(
                (1, _TS, dim),
                lambda i: ((jnp.maximum(i - nw, 0)) // ns,
                           (jnp.maximum(i - nw, 0)) % ns, 0)),
        ],
        out_specs=[
            pl.BlockSpec((1, 1, seq),
                         lambda i: ((jnp.maximum(i - nw, 0)) // ns, 0, 0)),
            pl.BlockSpec((1, 1, dim),
                         lambda i: ((jnp.maximum(i - nw, 0)) // ns, 0, 0)),
        ],
        out_shape=[
            jax.ShapeDtypeStruct((batch, 1, seq), jnp.float32),
            jax.ShapeDtypeStruct((batch, 1, dim), jnp.float32),
        ],
        scratch_shapes=[
            pltpu.VMEM((batch, dim), jnp.float32),
            pltpu.SMEM((2,), jnp.int32),
            pltpu.SMEM((2,), jnp.float32),
            pltpu.VMEM((1, dim), jnp.float32),
        ],
    )(lens, sel, h, W, context)
    return (weighted.reshape(batch, dim), attn.reshape(batch, seq))
